# baseline (speedup 1.0000x reference)
import functools

import jax
import jax.numpy as jnp
from jax import lax
from jax.experimental import pallas as pl
from jax.experimental.pallas import tpu as pltpu
from jax.experimental.pallas import tpu_sc as plsc

VOCAB = 1000000
DIM = 64
B = 16384
L = 200

NC = 2
NS = 16
NW = NC * NS

N = B * L
KSLICE = 2
NS_ROWS = N // KSLICE
ROWS_PER_W = NS_ROWS // NW
IDX_MINOR = 128
IDX_ROWS = 2
CHUNK = IDX_MINOR * IDX_ROWS
NBUF = 4
NCHUNKS = ROWS_PER_W // CHUNK


def _sc_gather(h2d, table):
    mesh = plsc.VectorSubcoreMesh(core_axis_name="c", subcore_axis_name="s")

    @functools.partial(
        pl.kernel,
        out_type=jax.ShapeDtypeStruct((NS_ROWS, DIM), jnp.float32),
        mesh=mesh,
        compiler_params=pltpu.CompilerParams(use_tc_tiling_on_sc=False),
        scratch_types=(
            [pltpu.VMEM((IDX_ROWS, IDX_MINOR), jnp.int32)] * NBUF
            + [pltpu.VMEM((CHUNK, DIM), jnp.float32)] * NBUF
            + [pltpu.SemaphoreType.DMA] * (3 * NBUF)
        ),
    )
    def k(h_hbm, table_hbm, out_hbm, *bufs):
        idx = bufs[:NBUF]
        rows = bufs[NBUF:2 * NBUF]
        si = bufs[2 * NBUF:2 * NBUF + NBUF]
        sg = bufs[3 * NBUF:3 * NBUF + NBUF]
        so = bufs[4 * NBUF:]

        wid = lax.axis_index("s") * NC + lax.axis_index("c")
        base = wid * ROWS_PER_W

        def idx_cp(c, b):
            irow0 = pl.multiple_of((base + c * CHUNK) // IDX_MINOR, IDX_ROWS)
            return pltpu.make_async_copy(
                h_hbm.at[pl.ds(irow0, IDX_ROWS)], idx[b], si[b])

        def gather_cps(b):
            return [
                pltpu.make_async_copy(
                    table_hbm.at[idx[b].at[j]],
                    rows[b].at[pl.ds(j * IDX_MINOR, IDX_MINOR)],
                    sg[b],
                )
                for j in range(IDX_ROWS)
            ]

        def out_cp(c, b):
            row0 = pl.multiple_of(base + c * CHUNK, CHUNK)
            return pltpu.make_async_copy(
                rows[b], out_hbm.at[pl.ds(row0, CHUNK)], so[b])

        for b in range(NBUF):
            idx_cp(b, b).start()
        for b in range(NBUF - 1):
            idx_cp(b, b).wait()
            for cp in gather_cps(b):
                cp.start()

        @pl.loop(0, NCHUNKS, step=NBUF)
        def _group(g):
            for b in range(NBUF):
                c = g + b
                a = (b + NBUF - 1) % NBUF

                @pl.when(c + NBUF - 1 < NCHUNKS)
                def _():
                    idx_cp(c + NBUF - 1, a).wait()

                    @pl.when(c >= 1)
                    def _():
                        out_cp(c - 1, a).wait()

                    for cp in gather_cps(a):
                        cp.start()

                for cp in gather_cps(b):
                    cp.wait()
                out_cp(c, b).start()

                @pl.when(c + NBUF < NCHUNKS)
                def _():
                    idx_cp(c + NBUF, b).start()

        for d in range(NBUF, 0, -1):
            c = NCHUNKS - d
            out_cp(c, c % NBUF).wait()

    return k(h2d, table)


def kernel(h, table):
    h2d = h.reshape(N // IDX_MINOR, IDX_MINOR)
    rows_2d = NS_ROWS // IDX_MINOR
    outs = []
    for i in range(KSLICE):
        part = _sc_gather(
            lax.slice_in_dim(h2d, i * rows_2d, (i + 1) * rows_2d, axis=0),
            table)
        outs.append(part.reshape(B // KSLICE, L * DIM))
    return jnp.concatenate(outs, axis=0)


# speedup vs baseline: 1.2157x; 1.2157x over previous
import functools

import jax
import jax.numpy as jnp
from jax import lax
from jax.experimental import pallas as pl
from jax.experimental.pallas import tpu as pltpu
from jax.experimental.pallas import tpu_sc as plsc

VOCAB = 1000000
DIM = 64
B = 16384
L = 200

NC = 2
NS = 16
NW = NC * NS

N = B * L
ROWS_PER_W = N // NW
IDX_MINOR = 128
IDX_ROWS = 1
CHUNK = IDX_MINOR * IDX_ROWS
NBUF = 8
NCHUNKS = ROWS_PER_W // CHUNK


def _sc_gather(h2d, table):
    mesh = plsc.VectorSubcoreMesh(core_axis_name="c", subcore_axis_name="s")

    @functools.partial(
        pl.kernel,
        out_type=jax.ShapeDtypeStruct((N, DIM), jnp.float32),
        mesh=mesh,
        compiler_params=pltpu.CompilerParams(use_tc_tiling_on_sc=False),
        scratch_types=(
            [pltpu.VMEM((IDX_ROWS, IDX_MINOR), jnp.int32)] * NBUF
            + [pltpu.VMEM((CHUNK, DIM), jnp.float32)] * NBUF
            + [pltpu.SemaphoreType.DMA] * (3 * NBUF)
        ),
    )
    def k(h_hbm, table_hbm, out_hbm, *bufs):
        idx = bufs[:NBUF]
        rows = bufs[NBUF:2 * NBUF]
        si = bufs[2 * NBUF:2 * NBUF + NBUF]
        sg = bufs[3 * NBUF:3 * NBUF + NBUF]
        so = bufs[4 * NBUF:]

        wid = lax.axis_index("s") * NC + lax.axis_index("c")
        base = wid * ROWS_PER_W

        def idx_cp(c, b):
            irow0 = pl.multiple_of((base + c * CHUNK) // IDX_MINOR, IDX_ROWS)
            return pltpu.make_async_copy(
                h_hbm.at[pl.ds(irow0, IDX_ROWS)], idx[b], si[b])

        def gather_cps(b):
            return [
                pltpu.make_async_copy(
                    table_hbm.at[idx[b].at[j]],
                    rows[b].at[pl.ds(j * IDX_MINOR, IDX_MINOR)],
                    sg[b],
                )
                for j in range(IDX_ROWS)
            ]

        def out_cp(c, b):
            row0 = pl.multiple_of(base + c * CHUNK, CHUNK)
            return pltpu.make_async_copy(
                rows[b], out_hbm.at[pl.ds(row0, CHUNK)], so[b])

        for b in range(NBUF):
            idx_cp(b, b).start()
        for b in range(NBUF - 1):
            idx_cp(b, b).wait()
            for cp in gather_cps(b):
                cp.start()

        @pl.loop(0, NCHUNKS, step=NBUF)
        def _group(g):
            for b in range(NBUF):
                c = g + b
                a = (b + NBUF - 1) % NBUF

                @pl.when(c + NBUF - 1 < NCHUNKS)
                def _():
                    idx_cp(c + NBUF - 1, a).wait()

                    @pl.when(c >= 1)
                    def _():
                        out_cp(c - 1, a).wait()

                    for cp in gather_cps(a):
                        cp.start()

                for cp in gather_cps(b):
                    cp.wait()
                out_cp(c, b).start()

                @pl.when(c + NBUF < NCHUNKS)
                def _():
                    idx_cp(c + NBUF, b).start()

        for d in range(NBUF, 0, -1):
            c = NCHUNKS - d
            out_cp(c, c % NBUF).wait()

    return k(h2d, table)


def kernel(h, table):
    h2d = h.reshape(N // IDX_MINOR, IDX_MINOR)
    out = _sc_gather(h2d, table)
    return out.reshape(B, L * DIM)


# speedup vs baseline: 1.2330x; 1.0142x over previous
import functools

import jax
import jax.numpy as jnp
from jax import lax
from jax.experimental import pallas as pl
from jax.experimental.pallas import tpu as pltpu
from jax.experimental.pallas import tpu_sc as plsc

VOCAB = 1000000
DIM = 64
B = 16384
L = 200

NC = 2
NS = 16
NW = NC * NS

N = B * L
ROWS_PER_W = N // NW
IDX_MINOR = 128
IDX_ROWS = 2
CHUNK = IDX_MINOR * IDX_ROWS
NBUF = 4
NCHUNKS = ROWS_PER_W // CHUNK


def _sc_gather(h2d, table):
    mesh = plsc.VectorSubcoreMesh(core_axis_name="c", subcore_axis_name="s")

    @functools.partial(
        pl.kernel,
        out_type=jax.ShapeDtypeStruct((N, DIM), jnp.float32),
        mesh=mesh,
        compiler_params=pltpu.CompilerParams(use_tc_tiling_on_sc=False),
        scratch_types=(
            [pltpu.VMEM((IDX_ROWS, IDX_MINOR), jnp.int32)] * NBUF
            + [pltpu.VMEM((CHUNK, DIM), jnp.float32)] * NBUF
            + [pltpu.SemaphoreType.DMA] * (3 * NBUF)
        ),
    )
    def k(h_hbm, table_hbm, out_hbm, *bufs):
        idx = bufs[:NBUF]
        rows = bufs[NBUF:2 * NBUF]
        si = bufs[2 * NBUF:2 * NBUF + NBUF]
        sg = bufs[3 * NBUF:3 * NBUF + NBUF]
        so = bufs[4 * NBUF:]

        wid = lax.axis_index("s") * NC + lax.axis_index("c")
        base = wid * ROWS_PER_W

        def idx_cp(c, b):
            irow0 = pl.multiple_of((base + c * CHUNK) // IDX_MINOR, IDX_ROWS)
            return pltpu.make_async_copy(
                h_hbm.at[pl.ds(irow0, IDX_ROWS)], idx[b], si[b])

        def gather_cps(b):
            return [
                pltpu.make_async_copy(
                    table_hbm.at[idx[b].at[j]],
                    rows[b].at[pl.ds(j * IDX_MINOR, IDX_MINOR)],
                    sg[b],
                )
                for j in range(IDX_ROWS)
            ]

        def out_cp(c, b):
            row0 = pl.multiple_of(base + c * CHUNK, CHUNK)
            return pltpu.make_async_copy(
                rows[b], out_hbm.at[pl.ds(row0, CHUNK)], so[b])

        for b in range(NBUF):
            idx_cp(b, b).start()
        for b in range(NBUF - 1):
            idx_cp(b, b).wait()
            for cp in gather_cps(b):
                cp.start()

        @pl.loop(0, NCHUNKS, step=NBUF)
        def _group(g):
            for b in range(NBUF):
                c = g + b
                a = (b + NBUF - 1) % NBUF

                @pl.when(c + NBUF - 1 < NCHUNKS)
                def _():
                    idx_cp(c + NBUF - 1, a).wait()

                    @pl.when(c >= 1)
                    def _():
                        out_cp(c - 1, a).wait()

                    for cp in gather_cps(a):
                        cp.start()

                for cp in gather_cps(b):
                    cp.wait()
                out_cp(c, b).start()

                @pl.when(c + NBUF < NCHUNKS)
                def _():
                    idx_cp(c + NBUF, b).start()

        for d in range(NBUF, 0, -1):
            c = NCHUNKS - d
            out_cp(c, c % NBUF).wait()

    return k(h2d, table)


def kernel(h, table):
    h2d = h.reshape(N // IDX_MINOR, IDX_MINOR)
    out = _sc_gather(h2d, table)
    return out.reshape(B, L * DIM)
